# baseline (speedup 1.0000x reference)
import dataclasses
import functools

import jax
import jax.numpy as jnp
import numpy as np
from jax import lax
from jax.experimental import pallas as pl
from jax.experimental.pallas import tpu as pltpu
from jax.experimental.pallas import tpu_sc as plsc


def _pack_bf16_pairs(x):
    b16 = x.astype(jnp.bfloat16)
    lo = lax.bitcast_convert_type(b16[..., :64], jnp.uint16).astype(jnp.uint32)
    hi = lax.bitcast_convert_type(b16[..., 64:], jnp.uint16).astype(jnp.uint32)
    return lo | (hi << 16)

N = 10000
E = 320000
F = 128
K = 16

NC = 2
NS = 16
NW = NC * NS
EPW = E // NW
C = 40
NCHUNK = EPW // C
RPS = 640

_NBLK = 1000
_EBLK = 6400


def _dot_t(a, w):
    return lax.dot_general(a, w, (((1,), (1,)), ((), ())),
                           preferred_element_type=jnp.float32)


def _node_body(x_ref, we_ref, be_ref, wv_ref, bv_ref, he_ref, hv_ref):
    xa = jnp.maximum(x_ref[...], 0.0)
    he = _dot_t(xa, we_ref[...]) + be_ref[...]
    he_ref[...] = jnp.maximum(he, 0.0)
    hv = _dot_t(xa, wv_ref[...]) + bv_ref[...]
    hv_ref[...] = jnp.maximum(hv, 0.0)


def _node_call(x, We, be, Wv, bv):
    grid = (N // _NBLK,)
    return pl.pallas_call(
        _node_body,
        grid=grid,
        in_specs=[
            pl.BlockSpec((_NBLK, F), lambda i: (i, 0)),
            pl.BlockSpec((F, F), lambda i: (0, 0)),
            pl.BlockSpec((1, F), lambda i: (0, 0)),
            pl.BlockSpec((F, F), lambda i: (0, 0)),
            pl.BlockSpec((1, F), lambda i: (0, 0)),
        ],
        out_specs=[
            pl.BlockSpec((_NBLK, F), lambda i: (i, 0)),
            pl.BlockSpec((_NBLK, F), lambda i: (i, 0)),
        ],
        out_shape=[
            jax.ShapeDtypeStruct((N, F), jnp.float32),
            jax.ShapeDtypeStruct((N, F), jnp.float32),
        ],
    )(x, We, be.reshape(1, F), Wv, bv.reshape(1, F))


def _gate_body(eat_ref, wg_ref, gate_ref):
    g = lax.dot_general(
        eat_ref[...], wg_ref[...], (((0,), (1,)), ((), ())),
        preferred_element_type=jnp.float32)
    nw = _EBLK // 80
    gr = g.reshape(nw, 2, 40, F)
    wa = _pack_bf16_pairs(gr[:, 0])
    wb = _pack_bf16_pairs(gr[:, 1])
    packed = jnp.concatenate([wa, wb], axis=2)
    gate_ref[...] = lax.bitcast_convert_type(
        packed.reshape(_EBLK // 2, F), jnp.int32)


def _gate_call(edge_attr, WG):
    grid = (E // _EBLK,)
    return pl.pallas_call(
        _gate_body,
        grid=grid,
        in_specs=[
            pl.BlockSpec((K, _EBLK), lambda i: (0, i)),
            pl.BlockSpec((F, K), lambda i: (0, 0)),
        ],
        out_specs=pl.BlockSpec((_EBLK // 2, F), lambda i: (i, 0)),
        out_shape=jax.ShapeDtypeStruct((E // 2, F), jnp.int32),
    )(edge_attr.T, WG)


def _edge_sc_body(h_hbm, gate_hbm, src_hbm, dst_hbm, out_hbm,
                  src_all, dst_all, rows0, rows1, rows2, rows3, gate0, gate1,
                  acc_sh, sem_g0, sem_g1, sem_r0, sem_r1, sem_r2, sem_r3,
                  sem_s0, sem_s1, sem_s2, sem_s3):
    core = lax.axis_index("core")
    sid = lax.axis_index("subcore")
    wid = sid * NC + core

    rows = (rows0, rows1, rows2, rows3)
    gate = (gate0, gate1)
    sem_g = (sem_g0, sem_g1)
    sem_r = (sem_r0, sem_r1, sem_r2, sem_r3)
    sem_s = (sem_s0, sem_s1, sem_s2, sem_s3)

    for buf in rows:
        @pl.loop(0, C)
        def _(r, buf=buf):
            for j in range(F // 16):
                buf[r, pl.ds(j * 16, 16)] = jnp.zeros((16,), jnp.float32)

    for kk in range(RPS // C):
        off = sid * RPS + kk * C

        @pl.when(off + C <= N)
        def _(buf=rows[kk % 4], off=off):
            pltpu.sync_copy(buf, acc_sh.at[pl.ds(off, C)])

    pltpu.sync_copy(src_hbm.at[pl.ds(wid * EPW, EPW)], src_all)
    pltpu.sync_copy(dst_hbm.at[pl.ds(wid * EPW, EPW)], dst_all)

    plsc.subcore_barrier()

    def issue(i, b, drain):
        if drain:
            pltpu.make_async_copy(rows[b],
                                  acc_sh.at[dst_all.at[pl.ds(0, C)]],
                                  sem_s[b]).wait()
        if b % 2 == 0:
            p = b // 2
            gbase = wid * (EPW // 2) + (i // 2) * C
            pltpu.async_copy(gate_hbm.at[pl.ds(gbase, C)], gate[p], sem_g[p])
        pltpu.async_copy(h_hbm.at[src_all.at[pl.ds(i * C, C)]], rows[b],
                         sem_r[b])

    def consume(i, b):
        p = b // 2
        goff = (b % 2) * (F // 2)
        if b % 2 == 0:
            pltpu.make_async_copy(gate_hbm.at[pl.ds(0, C)], gate[p],
                                  sem_g[p]).wait()
        pltpu.make_async_copy(h_hbm.at[src_all.at[pl.ds(0, C)]], rows[b],
                              sem_r[b]).wait()

        @pl.loop(0, C, step=2)
        def _(r):
            for rr in range(2):
                for j in range(F // 32):
                    gw = plsc.bitcast(
                        gate[p][r + rr, pl.ds(goff + j * 16, 16)],
                        jnp.bfloat16)
                    ga, gb = plsc.unpack(gw, format=plsc.PackFormat.INTERLEAVED)
                    sl_a = pl.ds(j * 16, 16)
                    sl_b = pl.ds((j + 4) * 16, 16)
                    rows[b][r + rr, sl_a] = rows[b][r + rr, sl_a] * ga
                    rows[b][r + rr, sl_b] = rows[b][r + rr, sl_b] * gb

        pltpu.async_copy(rows[b], acc_sh.at[dst_all.at[pl.ds(i * C, C)]],
                         sem_s[b], add=True)

    def step(j, with_issue):
        consume(j, j % 4)
        if with_issue:
            issue(j + 2, (j + 2) % 4, drain=j >= 2)

    issue(0, 0, False)
    issue(1, 1, False)
    step(0, True)
    step(1, True)
    step(2, True)
    step(3, True)

    @pl.loop(1, (NCHUNK - 8) // 4 + 1)
    def _(t):
        for k in range(4):
            j = 4 * t + k
            consume(j, k)
            issue(j + 2, (k + 2) % 4, drain=True)

    step(NCHUNK - 6, True)
    step(NCHUNK - 5, True)
    step(NCHUNK - 4, True)
    step(NCHUNK - 3, True)
    step(NCHUNK - 2, False)
    step(NCHUNK - 1, False)

    for b in range(4):
        pltpu.make_async_copy(rows[b], acc_sh.at[dst_all.at[pl.ds(0, C)]],
                              sem_s[b]).wait()

    plsc.subcore_barrier()

    @pl.when(sid < NS - 1)
    def _():
        pltpu.sync_copy(acc_sh.at[pl.ds(sid * RPS, RPS)],
                        out_hbm.at[core, pl.ds(sid * RPS, RPS)])

    @pl.when(sid == NS - 1)
    def _():
        pltpu.sync_copy(acc_sh.at[pl.ds((NS - 1) * RPS, N - (NS - 1) * RPS)],
                        out_hbm.at[core, pl.ds((NS - 1) * RPS,
                                               N - (NS - 1) * RPS)])


def _edge_sc_call(h_e, gate, src, dst):
    mesh = plsc.VectorSubcoreMesh(core_axis_name="core",
                                  subcore_axis_name="subcore")
    cp = pltpu.CompilerParams()
    if "needs_layout_passes" in pltpu.CompilerParams.__dataclass_fields__:
        cp = dataclasses.replace(cp, needs_layout_passes=False)
    k = pl.kernel(
        _edge_sc_body,
        out_type=jax.ShapeDtypeStruct((NC, N, F), jnp.float32),
        mesh=mesh,
        compiler_params=cp,
        scratch_types=[
            pltpu.VMEM((EPW,), jnp.int32),
            pltpu.VMEM((EPW,), jnp.int32),
            pltpu.VMEM((C, F), jnp.float32),
            pltpu.VMEM((C, F), jnp.float32),
            pltpu.VMEM((C, F), jnp.float32),
            pltpu.VMEM((C, F), jnp.float32),
            pltpu.VMEM((C, F), jnp.int32),
            pltpu.VMEM((C, F), jnp.int32),
            pltpu.VMEM_SHARED((N, F), jnp.float32),
            pltpu.SemaphoreType.DMA,
            pltpu.SemaphoreType.DMA,
            pltpu.SemaphoreType.DMA,
            pltpu.SemaphoreType.DMA,
            pltpu.SemaphoreType.DMA,
            pltpu.SemaphoreType.DMA,
            pltpu.SemaphoreType.DMA,
            pltpu.SemaphoreType.DMA,
            pltpu.SemaphoreType.DMA,
            pltpu.SemaphoreType.DMA,
        ],
    )
    return k(h_e, gate, src, dst)


def _post_body(p_ref, hv_ref, x_ref, u_ref, wr1_ref, br1_ref, wr2_ref,
               br2_ref, wout_ref, bout_ref, out1_ref, out2_ref):
    aggr = p_ref[0] + p_ref[1]
    msgx = hv_ref[...] + aggr
    out2_ref[...] = msgx
    tmp = msgx
    for i in range(2):
        h = jnp.maximum(tmp, 0.0)
        h = jnp.maximum(_dot_t(h, wr1_ref[i]) + br1_ref[i], 0.0)
        h = _dot_t(h, wr2_ref[i]) + br2_ref[i]
        tmp = tmp + h
    v = _dot_t(tmp, wout_ref[...]) + bout_ref[...]
    out1_ref[...] = v + x_ref[...] * u_ref[...]


def _post_call(partials, h_v, x, u, Wr1, br1, Wr2, br2, Wout, bout):
    grid = (N // _NBLK,)
    return pl.pallas_call(
        _post_body,
        grid=grid,
        in_specs=[
            pl.BlockSpec((NC, _NBLK, F), lambda i: (0, i, 0)),
            pl.BlockSpec((_NBLK, F), lambda i: (i, 0)),
            pl.BlockSpec((_NBLK, F), lambda i: (i, 0)),
            pl.BlockSpec((1, F), lambda i: (0, 0)),
            pl.BlockSpec((2, F, F), lambda i: (0, 0, 0)),
            pl.BlockSpec((2, 1, F), lambda i: (0, 0, 0)),
            pl.BlockSpec((2, F, F), lambda i: (0, 0, 0)),
            pl.BlockSpec((2, 1, F), lambda i: (0, 0, 0)),
            pl.BlockSpec((F, F), lambda i: (0, 0)),
            pl.BlockSpec((1, F), lambda i: (0, 0)),
        ],
        out_specs=[
            pl.BlockSpec((_NBLK, F), lambda i: (i, 0)),
            pl.BlockSpec((_NBLK, F), lambda i: (i, 0)),
        ],
        out_shape=[
            jax.ShapeDtypeStruct((N, F), jnp.float32),
            jax.ShapeDtypeStruct((N, F), jnp.float32),
        ],
    )(partials, h_v, x, u, Wr1, br1.reshape(2, 1, F), Wr2,
      br2.reshape(2, 1, F), Wout, bout.reshape(1, F))


def kernel(x, edge_index, edge_attr, Wv, bv, We, be, WG, u, Wr1, br1, Wr2,
           br2, Wout, bout):
    h_e, h_v = _node_call(x, We, be, Wv, bv)
    gate = _gate_call(edge_attr, WG)
    partials = _edge_sc_call(h_e, gate, edge_index[0], edge_index[1])
    out1, msgx = _post_call(partials, h_v, x, u, Wr1, br1, Wr2, br2, Wout,
                            bout)
    return (out1, msgx)


# speedup vs baseline: 1.0124x; 1.0124x over previous
import dataclasses
import functools

import jax
import jax.numpy as jnp
import numpy as np
from jax import lax
from jax.experimental import pallas as pl
from jax.experimental.pallas import tpu as pltpu
from jax.experimental.pallas import tpu_sc as plsc


def _rne_bf16_bits(u):
    return (u + jnp.uint32(0x7FFF) + ((u >> 16) & jnp.uint32(1))) >> 16


def _pack_bf16_pairs(x):
    return x[..., :64] | (x[..., 64:] << 16)

N = 10000
E = 320000
F = 128
K = 16

NC = 2
NS = 16
NW = NC * NS
EPW = E // NW
C = 40
NCHUNK = EPW // C
RPS = 640

_NBLK = 1000
_EBLK = 6400


def _dot_t(a, w):
    return lax.dot_general(a, w, (((1,), (1,)), ((), ())),
                           preferred_element_type=jnp.float32)


def _node_body(x_ref, we_ref, be_ref, wv_ref, bv_ref, he_ref, hv_ref):
    xa = jnp.maximum(x_ref[...], 0.0)
    he = _dot_t(xa, we_ref[...]) + be_ref[...]
    he_ref[...] = jnp.maximum(he, 0.0)
    hv = _dot_t(xa, wv_ref[...]) + bv_ref[...]
    hv_ref[...] = jnp.maximum(hv, 0.0)


def _node_call(x, We, be, Wv, bv):
    grid = (N // _NBLK,)
    return pl.pallas_call(
        _node_body,
        grid=grid,
        in_specs=[
            pl.BlockSpec((_NBLK, F), lambda i: (i, 0)),
            pl.BlockSpec((F, F), lambda i: (0, 0)),
            pl.BlockSpec((1, F), lambda i: (0, 0)),
            pl.BlockSpec((F, F), lambda i: (0, 0)),
            pl.BlockSpec((1, F), lambda i: (0, 0)),
        ],
        out_specs=[
            pl.BlockSpec((_NBLK, F), lambda i: (i, 0)),
            pl.BlockSpec((_NBLK, F), lambda i: (i, 0)),
        ],
        out_shape=[
            jax.ShapeDtypeStruct((N, F), jnp.float32),
            jax.ShapeDtypeStruct((N, F), jnp.float32),
        ],
    )(x, We, be.reshape(1, F), Wv, bv.reshape(1, F))


def _gate_body(eat_ref, wg_ref, gate_ref):
    g = lax.dot_general(
        eat_ref[...], wg_ref[...], (((0,), (1,)), ((), ())),
        preferred_element_type=jnp.float32)
    rb = _rne_bf16_bits(lax.bitcast_convert_type(g, jnp.uint32))
    nw = _EBLK // 80
    gr = rb.reshape(nw, 2, 40, F)
    wa = _pack_bf16_pairs(gr[:, 0])
    wb = _pack_bf16_pairs(gr[:, 1])
    packed = jnp.concatenate([wa, wb], axis=2)
    gate_ref[...] = lax.bitcast_convert_type(
        packed.reshape(_EBLK // 2, F), jnp.int32)


def _gate_call(edge_attr, WG):
    grid = (E // _EBLK,)
    return pl.pallas_call(
        _gate_body,
        grid=grid,
        in_specs=[
            pl.BlockSpec((K, _EBLK), lambda i: (0, i)),
            pl.BlockSpec((F, K), lambda i: (0, 0)),
        ],
        out_specs=pl.BlockSpec((_EBLK // 2, F), lambda i: (i, 0)),
        out_shape=jax.ShapeDtypeStruct((E // 2, F), jnp.int32),
    )(edge_attr.T, WG)


def _edge_sc_body(h_hbm, gate_hbm, ei_hbm, out_hbm,
                  src_all, dst_all, rows0, rows1, rows2, rows3, gate0, gate1,
                  acc_sh, sem_g0, sem_g1, sem_r0, sem_r1, sem_r2, sem_r3,
                  sem_s0, sem_s1, sem_s2, sem_s3):
    core = lax.axis_index("core")
    sid = lax.axis_index("subcore")
    wid = sid * NC + core

    rows = (rows0, rows1, rows2, rows3)
    gate = (gate0, gate1)
    sem_g = (sem_g0, sem_g1)
    sem_r = (sem_r0, sem_r1, sem_r2, sem_r3)
    sem_s = (sem_s0, sem_s1, sem_s2, sem_s3)

    for buf in rows:
        @pl.loop(0, C)
        def _(r, buf=buf):
            for j in range(F // 16):
                buf[r, pl.ds(j * 16, 16)] = jnp.zeros((16,), jnp.float32)

    for kk in range(RPS // C):
        off = sid * RPS + kk * C

        @pl.when(off + C <= N)
        def _(buf=rows[kk % 4], off=off):
            pltpu.sync_copy(buf, acc_sh.at[pl.ds(off, C)])

    pltpu.sync_copy(ei_hbm.at[pl.ds(wid * EPW, EPW)], src_all)
    pltpu.sync_copy(ei_hbm.at[pl.ds(E + wid * EPW, EPW)], dst_all)

    plsc.subcore_barrier()

    def issue(i, b, drain):
        if drain:
            pltpu.make_async_copy(rows[b],
                                  acc_sh.at[dst_all.at[pl.ds(0, C)]],
                                  sem_s[b]).wait()
        if b % 2 == 0:
            p = b // 2
            gbase = wid * (EPW // 2) + (i // 2) * C
            pltpu.async_copy(gate_hbm.at[pl.ds(gbase, C)], gate[p], sem_g[p])
        pltpu.async_copy(h_hbm.at[src_all.at[pl.ds(i * C, C)]], rows[b],
                         sem_r[b])

    def consume(i, b):
        p = b // 2
        goff = (b % 2) * (F // 2)
        if b % 2 == 0:
            pltpu.make_async_copy(gate_hbm.at[pl.ds(0, C)], gate[p],
                                  sem_g[p]).wait()
        pltpu.make_async_copy(h_hbm.at[src_all.at[pl.ds(0, C)]], rows[b],
                              sem_r[b]).wait()

        @pl.loop(0, C, step=4)
        def _(r):
            for rr in range(4):
                for j in range(F // 32):
                    gw = plsc.bitcast(
                        gate[p][r + rr, pl.ds(goff + j * 16, 16)],
                        jnp.bfloat16)
                    ga, gb = plsc.unpack(gw, format=plsc.PackFormat.INTERLEAVED)
                    sl_a = pl.ds(j * 16, 16)
                    sl_b = pl.ds((j + 4) * 16, 16)
                    rows[b][r + rr, sl_a] = rows[b][r + rr, sl_a] * ga
                    rows[b][r + rr, sl_b] = rows[b][r + rr, sl_b] * gb

        pltpu.async_copy(rows[b], acc_sh.at[dst_all.at[pl.ds(i * C, C)]],
                         sem_s[b], add=True)

    def step(j, with_issue):
        consume(j, j % 4)
        if with_issue:
            issue(j + 2, (j + 2) % 4, drain=j >= 2)

    issue(0, 0, False)
    issue(1, 1, False)
    step(0, True)
    step(1, True)
    step(2, True)
    step(3, True)

    @pl.loop(1, (NCHUNK - 8) // 4 + 1)
    def _(t):
        for k in range(4):
            j = 4 * t + k
            consume(j, k)
            issue(j + 2, (k + 2) % 4, drain=True)

    step(NCHUNK - 6, True)
    step(NCHUNK - 5, True)
    step(NCHUNK - 4, True)
    step(NCHUNK - 3, True)
    step(NCHUNK - 2, False)
    step(NCHUNK - 1, False)

    for b in range(4):
        pltpu.make_async_copy(rows[b], acc_sh.at[dst_all.at[pl.ds(0, C)]],
                              sem_s[b]).wait()

    plsc.subcore_barrier()

    @pl.when(sid < NS - 1)
    def _():
        pltpu.sync_copy(acc_sh.at[pl.ds(sid * RPS, RPS)],
                        out_hbm.at[core, pl.ds(sid * RPS, RPS)])

    @pl.when(sid == NS - 1)
    def _():
        pltpu.sync_copy(acc_sh.at[pl.ds((NS - 1) * RPS, N - (NS - 1) * RPS)],
                        out_hbm.at[core, pl.ds((NS - 1) * RPS,
                                               N - (NS - 1) * RPS)])


def _edge_sc_call(h_e, gate, ei_flat):
    mesh = plsc.VectorSubcoreMesh(core_axis_name="core",
                                  subcore_axis_name="subcore")
    cp = pltpu.CompilerParams()
    if "needs_layout_passes" in pltpu.CompilerParams.__dataclass_fields__:
        cp = dataclasses.replace(cp, needs_layout_passes=False)
    k = pl.kernel(
        _edge_sc_body,
        out_type=jax.ShapeDtypeStruct((NC, N, F), jnp.float32),
        mesh=mesh,
        compiler_params=cp,
        scratch_types=[
            pltpu.VMEM((EPW,), jnp.int32),
            pltpu.VMEM((EPW,), jnp.int32),
            pltpu.VMEM((C, F), jnp.float32),
            pltpu.VMEM((C, F), jnp.float32),
            pltpu.VMEM((C, F), jnp.float32),
            pltpu.VMEM((C, F), jnp.float32),
            pltpu.VMEM((C, F), jnp.int32),
            pltpu.VMEM((C, F), jnp.int32),
            pltpu.VMEM_SHARED((N, F), jnp.float32),
            pltpu.SemaphoreType.DMA,
            pltpu.SemaphoreType.DMA,
            pltpu.SemaphoreType.DMA,
            pltpu.SemaphoreType.DMA,
            pltpu.SemaphoreType.DMA,
            pltpu.SemaphoreType.DMA,
            pltpu.SemaphoreType.DMA,
            pltpu.SemaphoreType.DMA,
            pltpu.SemaphoreType.DMA,
            pltpu.SemaphoreType.DMA,
        ],
    )
    return k(h_e, gate, ei_flat)


def _post_body(p_ref, hv_ref, x_ref, u_ref, wr1_ref, br1_ref, wr2_ref,
               br2_ref, wout_ref, bout_ref, out1_ref, out2_ref):
    aggr = p_ref[0] + p_ref[1]
    msgx = hv_ref[...] + aggr
    out2_ref[...] = msgx
    tmp = msgx
    for i in range(2):
        h = jnp.maximum(tmp, 0.0)
        h = jnp.maximum(_dot_t(h, wr1_ref[i]) + br1_ref[i], 0.0)
        h = _dot_t(h, wr2_ref[i]) + br2_ref[i]
        tmp = tmp + h
    v = _dot_t(tmp, wout_ref[...]) + bout_ref[...]
    out1_ref[...] = v + x_ref[...] * u_ref[...]


def _post_call(partials, h_v, x, u, Wr1, br1, Wr2, br2, Wout, bout):
    grid = (N // _NBLK,)
    return pl.pallas_call(
        _post_body,
        grid=grid,
        in_specs=[
            pl.BlockSpec((NC, _NBLK, F), lambda i: (0, i, 0)),
            pl.BlockSpec((_NBLK, F), lambda i: (i, 0)),
            pl.BlockSpec((_NBLK, F), lambda i: (i, 0)),
            pl.BlockSpec((1, F), lambda i: (0, 0)),
            pl.BlockSpec((2, F, F), lambda i: (0, 0, 0)),
            pl.BlockSpec((2, 1, F), lambda i: (0, 0, 0)),
            pl.BlockSpec((2, F, F), lambda i: (0, 0, 0)),
            pl.BlockSpec((2, 1, F), lambda i: (0, 0, 0)),
            pl.BlockSpec((F, F), lambda i: (0, 0)),
            pl.BlockSpec((1, F), lambda i: (0, 0)),
        ],
        out_specs=[
            pl.BlockSpec((_NBLK, F), lambda i: (i, 0)),
            pl.BlockSpec((_NBLK, F), lambda i: (i, 0)),
        ],
        out_shape=[
            jax.ShapeDtypeStruct((N, F), jnp.float32),
            jax.ShapeDtypeStruct((N, F), jnp.float32),
        ],
    )(partials, h_v, x, u, Wr1, br1.reshape(2, 1, F), Wr2,
      br2.reshape(2, 1, F), Wout, bout.reshape(1, F))


def kernel(x, edge_index, edge_attr, Wv, bv, We, be, WG, u, Wr1, br1, Wr2,
           br2, Wout, bout):
    h_e, h_v = _node_call(x, We, be, Wv, bv)
    gate = _gate_call(edge_attr, WG)
    partials = _edge_sc_call(h_e, gate, edge_index.reshape(2 * E))
    out1, msgx = _post_call(partials, h_v, x, u, Wr1, br1, Wr2, br2, Wout,
                            bout)
    return (out1, msgx)


# speedup vs baseline: 1.0325x; 1.0199x over previous
import dataclasses
import functools

import jax
import jax.numpy as jnp
import numpy as np
from jax import lax
from jax.experimental import pallas as pl
from jax.experimental.pallas import tpu as pltpu
from jax.experimental.pallas import tpu_sc as plsc

N = 10000
E = 320000
F = 128
K = 16

NC = 2
NS = 16
NW = NC * NS
EPW = E // NW
C = 40
NCHUNK = EPW // C
RPS = 640

_NBLK = 1000
_EBLK = 6400


def _dot_t(a, w):
    return lax.dot_general(a, w, (((1,), (1,)), ((), ())),
                           preferred_element_type=jnp.float32)


def _node_body(x_ref, we_ref, be_ref, wv_ref, bv_ref, he_ref, hv_ref):
    xa = jnp.maximum(x_ref[...], 0.0)
    he = _dot_t(xa, we_ref[...]) + be_ref[...]
    he_ref[...] = jnp.maximum(he, 0.0)
    hv = _dot_t(xa, wv_ref[...]) + bv_ref[...]
    hv_ref[...] = jnp.maximum(hv, 0.0)


def _node_call(x, We, be, Wv, bv):
    grid = (N // _NBLK,)
    return pl.pallas_call(
        _node_body,
        grid=grid,
        in_specs=[
            pl.BlockSpec((_NBLK, F), lambda i: (i, 0)),
            pl.BlockSpec((F, F), lambda i: (0, 0)),
            pl.BlockSpec((1, F), lambda i: (0, 0)),
            pl.BlockSpec((F, F), lambda i: (0, 0)),
            pl.BlockSpec((1, F), lambda i: (0, 0)),
        ],
        out_specs=[
            pl.BlockSpec((_NBLK, F), lambda i: (i, 0)),
            pl.BlockSpec((_NBLK, F), lambda i: (i, 0)),
        ],
        out_shape=[
            jax.ShapeDtypeStruct((N, F), jnp.float32),
            jax.ShapeDtypeStruct((N, F), jnp.float32),
        ],
    )(x, We, be.reshape(1, F), Wv, bv.reshape(1, F))


def _gate_body(eat_ref, wg_ref, gate_ref):
    gate_ref[...] = lax.dot_general(
        eat_ref[...], wg_ref[...], (((0,), (1,)), ((), ())),
        preferred_element_type=jnp.float32)


def _gate_call(edge_attr, WG):
    grid = (E // _EBLK,)
    return pl.pallas_call(
        _gate_body,
        grid=grid,
        in_specs=[
            pl.BlockSpec((K, _EBLK), lambda i: (0, i)),
            pl.BlockSpec((F, K), lambda i: (0, 0)),
        ],
        out_specs=pl.BlockSpec((_EBLK, F), lambda i: (i, 0)),
        out_shape=jax.ShapeDtypeStruct((E, F), jnp.float32),
    )(edge_attr.T, WG)


def _edge_sc_body(h_hbm, gate_hbm, ei_hbm, out_hbm,
                  src_all, dst_all, rows0, rows1, rows2, rows3, gate0, gate1,
                  acc_sh, sem_g0, sem_g1, sem_r0, sem_r1, sem_r2, sem_r3,
                  sem_s0, sem_s1, sem_s2, sem_s3):
    core = lax.axis_index("core")
    sid = lax.axis_index("subcore")
    wid = sid * NC + core

    rows = (rows0, rows1, rows2, rows3)
    gate = (gate0, gate1)
    sem_g = (sem_g0, sem_g1)
    sem_r = (sem_r0, sem_r1, sem_r2, sem_r3)
    sem_s = (sem_s0, sem_s1, sem_s2, sem_s3)

    for buf in rows:
        @pl.loop(0, C)
        def _(r, buf=buf):
            for j in range(F // 16):
                buf[r, pl.ds(j * 16, 16)] = jnp.zeros((16,), jnp.float32)

    for kk in range(RPS // C):
        off = sid * RPS + kk * C

        @pl.when(off + C <= N)
        def _(buf=rows[kk % 4], off=off):
            pltpu.sync_copy(buf, acc_sh.at[pl.ds(off, C)])

    pltpu.sync_copy(ei_hbm.at[pl.ds(wid * EPW, EPW)], src_all)
    pltpu.sync_copy(ei_hbm.at[pl.ds(E + wid * EPW, EPW)], dst_all)

    plsc.subcore_barrier()

    def issue(i, b, drain):
        if drain:
            pltpu.make_async_copy(rows[b],
                                  acc_sh.at[dst_all.at[pl.ds(0, C)]],
                                  sem_s[b]).wait()
        p = b % 2
        base = wid * EPW + i * C
        pltpu.async_copy(gate_hbm.at[pl.ds(base, C)], gate[p], sem_g[p])
        pltpu.async_copy(h_hbm.at[src_all.at[pl.ds(i * C, C)]], rows[b],
                         sem_r[b])

    def consume(i, b):
        p = b % 2
        pltpu.make_async_copy(gate_hbm.at[pl.ds(0, C)], gate[p],
                              sem_g[p]).wait()
        pltpu.make_async_copy(h_hbm.at[src_all.at[pl.ds(0, C)]], rows[b],
                              sem_r[b]).wait()

        @pl.loop(0, C, step=4)
        def _(r):
            for rr in range(4):
                for j in range(F // 16):
                    sl = pl.ds(j * 16, 16)
                    rows[b][r + rr, sl] = (rows[b][r + rr, sl]
                                           * gate[p][r + rr, sl])

        pltpu.async_copy(rows[b], acc_sh.at[dst_all.at[pl.ds(i * C, C)]],
                         sem_s[b], add=True)

    def step(j, with_issue):
        consume(j, j % 4)
        if with_issue:
            issue(j + 2, (j + 2) % 4, drain=j >= 2)

    issue(0, 0, False)
    issue(1, 1, False)
    step(0, True)
    step(1, True)
    step(2, True)
    step(3, True)

    @pl.loop(1, (NCHUNK - 8) // 4 + 1)
    def _(t):
        for k in range(4):
            j = 4 * t + k
            consume(j, k)
            issue(j + 2, (k + 2) % 4, drain=True)

    step(NCHUNK - 6, True)
    step(NCHUNK - 5, True)
    step(NCHUNK - 4, True)
    step(NCHUNK - 3, True)
    step(NCHUNK - 2, False)
    step(NCHUNK - 1, False)

    for b in range(4):
        pltpu.make_async_copy(rows[b], acc_sh.at[dst_all.at[pl.ds(0, C)]],
                              sem_s[b]).wait()

    plsc.subcore_barrier()

    @pl.when(sid < NS - 1)
    def _():
        pltpu.sync_copy(acc_sh.at[pl.ds(sid * RPS, RPS)],
                        out_hbm.at[core, pl.ds(sid * RPS, RPS)])

    @pl.when(sid == NS - 1)
    def _():
        pltpu.sync_copy(acc_sh.at[pl.ds((NS - 1) * RPS, N - (NS - 1) * RPS)],
                        out_hbm.at[core, pl.ds((NS - 1) * RPS,
                                               N - (NS - 1) * RPS)])


def _edge_sc_call(h_e, gate, ei_flat):
    mesh = plsc.VectorSubcoreMesh(core_axis_name="core",
                                  subcore_axis_name="subcore")
    cp = pltpu.CompilerParams()
    if "needs_layout_passes" in pltpu.CompilerParams.__dataclass_fields__:
        cp = dataclasses.replace(cp, needs_layout_passes=False)
    k = pl.kernel(
        _edge_sc_body,
        out_type=jax.ShapeDtypeStruct((NC, N, F), jnp.float32),
        mesh=mesh,
        compiler_params=cp,
        scratch_types=[
            pltpu.VMEM((EPW,), jnp.int32),
            pltpu.VMEM((EPW,), jnp.int32),
            pltpu.VMEM((C, F), jnp.float32),
            pltpu.VMEM((C, F), jnp.float32),
            pltpu.VMEM((C, F), jnp.float32),
            pltpu.VMEM((C, F), jnp.float32),
            pltpu.VMEM((C, F), jnp.float32),
            pltpu.VMEM((C, F), jnp.float32),
            pltpu.VMEM_SHARED((N, F), jnp.float32),
            pltpu.SemaphoreType.DMA,
            pltpu.SemaphoreType.DMA,
            pltpu.SemaphoreType.DMA,
            pltpu.SemaphoreType.DMA,
            pltpu.SemaphoreType.DMA,
            pltpu.SemaphoreType.DMA,
            pltpu.SemaphoreType.DMA,
            pltpu.SemaphoreType.DMA,
            pltpu.SemaphoreType.DMA,
            pltpu.SemaphoreType.DMA,
        ],
    )
    return k(h_e, gate, ei_flat)


def _post_body(p_ref, hv_ref, x_ref, u_ref, wr1_ref, br1_ref, wr2_ref,
               br2_ref, wout_ref, bout_ref, out1_ref, out2_ref):
    aggr = p_ref[0] + p_ref[1]
    msgx = hv_ref[...] + aggr
    out2_ref[...] = msgx
    tmp = msgx
    for i in range(2):
        h = jnp.maximum(tmp, 0.0)
        h = jnp.maximum(_dot_t(h, wr1_ref[i]) + br1_ref[i], 0.0)
        h = _dot_t(h, wr2_ref[i]) + br2_ref[i]
        tmp = tmp + h
    v = _dot_t(tmp, wout_ref[...]) + bout_ref[...]
    out1_ref[...] = v + x_ref[...] * u_ref[...]


def _post_call(partials, h_v, x, u, Wr1, br1, Wr2, br2, Wout, bout):
    grid = (N // _NBLK,)
    return pl.pallas_call(
        _post_body,
        grid=grid,
        in_specs=[
            pl.BlockSpec((NC, _NBLK, F), lambda i: (0, i, 0)),
            pl.BlockSpec((_NBLK, F), lambda i: (i, 0)),
            pl.BlockSpec((_NBLK, F), lambda i: (i, 0)),
            pl.BlockSpec((1, F), lambda i: (0, 0)),
            pl.BlockSpec((2, F, F), lambda i: (0, 0, 0)),
            pl.BlockSpec((2, 1, F), lambda i: (0, 0, 0)),
            pl.BlockSpec((2, F, F), lambda i: (0, 0, 0)),
            pl.BlockSpec((2, 1, F), lambda i: (0, 0, 0)),
            pl.BlockSpec((F, F), lambda i: (0, 0)),
            pl.BlockSpec((1, F), lambda i: (0, 0)),
        ],
        out_specs=[
            pl.BlockSpec((_NBLK, F), lambda i: (i, 0)),
            pl.BlockSpec((_NBLK, F), lambda i: (i, 0)),
        ],
        out_shape=[
            jax.ShapeDtypeStruct((N, F), jnp.float32),
            jax.ShapeDtypeStruct((N, F), jnp.float32),
        ],
    )(partials, h_v, x, u, Wr1, br1.reshape(2, 1, F), Wr2,
      br2.reshape(2, 1, F), Wout, bout.reshape(1, F))


def kernel(x, edge_index, edge_attr, Wv, bv, We, be, WG, u, Wr1, br1, Wr2,
           br2, Wout, bout):
    h_e, h_v = _node_call(x, We, be, Wv, bv)
    gate = _gate_call(edge_attr, WG)
    partials = _edge_sc_call(h_e, gate, edge_index.reshape(2 * E))
    out1, msgx = _post_call(partials, h_v, x, u, Wr1, br1, Wr2, br2, Wout,
                            bout)
    return (out1, msgx)
